# baseline (speedup 1.0000x reference)
import functools

import jax
import jax.numpy as jnp
from jax import lax
from jax.experimental import pallas as pl
from jax.experimental.pallas import tpu as pltpu
from jax.experimental.pallas import tpu_sc as plsc

B = 4096
L = 50
EMB = 32
NC = 2
NS = 16
NW = NC * NS
SPW = B // NW
GS = 16
NG = SPW // GS
ROWS_G = GS * L


def _body(did, watch, vid, prev, region, mod, mf, sver, aver, cid,
          class_id, second_class, is_intact,
          uemb_t, vemb_t, remb_t, mod_t, mf_t, sver_t, aver_t,
          cemb_t, class_t, second_t, intact_t, w, b,
          out,
          widx, wbuf,
          f_u, f_v, f_p, f_c, f_r, f_mod, f_mf, f_sv, f_av, f_cl, f_sc, f_in,
          i_u, i_v, i_p, i_c, i_r, i_mod, i_mf, i_sv, i_av, i_cl, i_sc, i_in,
          w_v, b_v, row0, zb, qb, ov, sem, semw):
    wid = lax.axis_index("s") * NC + lax.axis_index("c")
    base = wid * SPW

    pltpu.sync_copy(watch.at[pl.ds(base * L, SPW * L)], widx)
    for src, dst in ((did, i_u), (vid, i_v), (prev, i_p), (cid, i_c),
                     (region, i_r), (mod, i_mod), (mf, i_mf), (sver, i_sv),
                     (aver, i_av), (class_id, i_cl), (second_class, i_sc),
                     (is_intact, i_in)):
        pltpu.sync_copy(src.at[pl.ds(base, SPW)], dst)
    pltpu.sync_copy(w, w_v)
    pltpu.sync_copy(b, b_v)
    pltpu.sync_copy(vemb_t.at[pl.ds(0, 1)], row0)

    copies = [pltpu.async_copy(tbl.at[idx], buf, sem)
              for tbl, idx, buf in ((uemb_t, i_u, f_u), (vemb_t, i_v, f_v),
                                    (vemb_t, i_p, f_p), (cemb_t, i_c, f_c),
                                    (remb_t, i_r, f_r), (mod_t, i_mod, f_mod),
                                    (mf_t, i_mf, f_mf), (sver_t, i_sv, f_sv),
                                    (aver_t, i_av, f_av), (class_t, i_cl, f_cl),
                                    (second_t, i_sc, f_sc),
                                    (intact_t, i_in, f_in))]
    for c in copies:
        c.wait()

    wv = [(w_v[pl.ds(32 * k, 16)], w_v[pl.ds(32 * k + 16, 16)])
          for k in range(8)]
    iota = lax.iota(jnp.int32, 16)
    bvec = b_v[...]
    qb[...] = (row0[0, pl.ds(0, 16)] * wv[1][0]
               + row0[0, pl.ds(16, 16)] * wv[1][1])

    def rr(j, acc):
        return acc + plsc.load_gather(qb, [iota * 0 + j])

    rhov = lax.fori_loop(0, 16, rr, jnp.zeros((16,), jnp.float32))
    feats = ((f_u, 0), (f_v, 3), (f_p, 5), (f_c, 6), (f_cl, 6), (f_sc, 6),
             (f_in, 6), (f_r, 7), (f_mod, 2), (f_mf, 2), (f_sv, 2),
             (f_av, 2))

    def group(g, _):
        pltpu.async_copy(vemb_t.at[widx.at[pl.ds(g * ROWS_G, ROWS_G)]],
                         wbuf, semw).wait()
        def cbody(s, cnt):
            v = plsc.load_gather(widx, [g * ROWS_G + iota * L + s])
            return cnt + jnp.where(v == 0, 1, 0)
        cnt = lax.fori_loop(0, L, cbody, jnp.zeros((16,), jnp.int32))

        def sbody(i, _):
            sg = g * GS + i

            def wb(s, zz):
                z0, z1 = zz
                r = i * L + s
                return (z0 + wbuf[r, pl.ds(0, 16)] * wv[1][0],
                        z1 + wbuf[r, pl.ds(16, 16)] * wv[1][1])

            z0, z1 = lax.fori_loop(0, L, wb,
                                   (jnp.zeros((16,), jnp.float32),
                                    jnp.zeros((16,), jnp.float32)))
            z0 = z0 + wbuf[i * L, pl.ds(0, 16)] * wv[4][0]
            z1 = z1 + wbuf[i * L, pl.ds(16, 16)] * wv[4][1]
            for buf, k in feats:
                z0 = z0 + buf[sg, pl.ds(0, 16)] * wv[k][0]
                z1 = z1 + buf[sg, pl.ds(16, 16)] * wv[k][1]
            zb[pl.ds(i * EMB, 16)] = z0
            zb[pl.ds(i * EMB + 16, 16)] = z1
            return 0

        lax.fori_loop(0, GS, sbody, 0)

        def rbody(j, s):
            return s + plsc.load_gather(zb, [iota * EMB + j])

        s = lax.fori_loop(0, EMB, rbody, jnp.zeros((16,), jnp.float32))
        ov[pl.ds(g * GS, GS)] = s - cnt.astype(jnp.float32) * rhov + bvec
        return 0

    lax.fori_loop(0, NG, group, 0)
    pltpu.sync_copy(ov, out.at[pl.ds(base, SPW)])


def kernel(did, watch_vids, vid, prev, region, mod, mf, sver, aver, cid,
           class_id, second_class, is_intact,
           uemb_table, vemb_table, remb_table, mod_table, mf_table,
           sver_table, aver_table, cemb_table, class_table,
           second_class_table, intact_table, W, b):
    i32 = lambda x: x.astype(jnp.int32)
    mesh = plsc.VectorSubcoreMesh(core_axis_name="c", subcore_axis_name="s")
    scratch = [
        pltpu.VMEM((SPW * L,), jnp.int32),
        pltpu.VMEM((ROWS_G, EMB), jnp.float32),
    ] + [pltpu.VMEM((SPW, EMB), jnp.float32)] * 12 \
      + [pltpu.VMEM((SPW,), jnp.int32)] * 12 + [
        pltpu.VMEM((8 * EMB,), jnp.float32),
        pltpu.VMEM((16,), jnp.float32),
        pltpu.VMEM((1, EMB), jnp.float32),
        pltpu.VMEM((GS * EMB,), jnp.float32),
        pltpu.VMEM((16,), jnp.float32),
        pltpu.VMEM((SPW,), jnp.float32),
        pltpu.SemaphoreType.DMA,
        pltpu.SemaphoreType.DMA,
    ]
    run = pl.kernel(_body,
                    out_type=jax.ShapeDtypeStruct((B,), jnp.float32),
                    mesh=mesh, scratch_types=scratch,
                    compiler_params=pltpu.CompilerParams(
                        needs_layout_passes=False,
                        use_tc_tiling_on_sc=False))
    return run(i32(did), i32(watch_vids).reshape(-1), i32(vid), i32(prev),
               i32(region), i32(mod), i32(mf), i32(sver), i32(aver),
               i32(cid), i32(class_id), i32(second_class), i32(is_intact),
               uemb_table, vemb_table, remb_table, mod_table, mf_table,
               sver_table, aver_table, cemb_table, class_table,
               second_class_table, intact_table,
               W.reshape(-1).astype(jnp.float32),
               jnp.broadcast_to(b.astype(jnp.float32).reshape(()), (16,)))


# speedup vs baseline: 1.0266x; 1.0266x over previous
import jax
import jax.numpy as jnp
from jax import lax
from jax.experimental import pallas as pl
from jax.experimental.pallas import tpu as pltpu
from jax.experimental.pallas import tpu_sc as plsc

B = 4096
L = 50
EMB = 32
NC = 2
NS = 16
NW = NC * NS
SPW = B // NW
GS = 16
NG = SPW // GS
ROWS_G = GS * L
NF = 12


def _body(watch, stk,
          uemb_t, vemb_t, remb_t, mod_t, mf_t, sver_t, aver_t,
          cemb_t, class_t, second_t, intact_t, w, b,
          out,
          widx, idxs, wbufa, wbufb,
          f_u, f_v, f_p, f_c, f_r, f_mod, f_mf, f_sv, f_av, f_cl, f_sc, f_in,
          w_v, b_v, row0, zb, qb, ov, sem_i, sem_f, sem_a, sem_b):
    wid = lax.axis_index("s") * NC + lax.axis_index("c")
    base = wid * SPW

    d1 = pltpu.async_copy(watch.at[pl.ds(base * L, SPW * L)], widx, sem_i)
    d2 = pltpu.async_copy(stk.at[:, pl.ds(base, SPW)], idxs, sem_i)
    pltpu.sync_copy(w, w_v)
    pltpu.sync_copy(b, b_v)
    pltpu.sync_copy(vemb_t.at[pl.ds(0, 1)], row0)
    d1.wait()
    d2.wait()

    fbufs = (f_u, f_v, f_p, f_c, f_r, f_mod, f_mf, f_sv, f_av, f_cl,
             f_sc, f_in)
    tables = (uemb_t, vemb_t, vemb_t, cemb_t, remb_t, mod_t, mf_t, sver_t,
              aver_t, class_t, second_t, intact_t)
    copies = [pltpu.async_copy(tbl.at[idxs.at[f]], buf, sem_f)
              for f, (tbl, buf) in enumerate(zip(tables, fbufs))]
    pltpu.async_copy(vemb_t.at[widx.at[pl.ds(0, ROWS_G)]], wbufa, sem_a)
    pltpu.async_copy(vemb_t.at[widx.at[pl.ds(ROWS_G, ROWS_G)]], wbufb, sem_b)
    for c in copies:
        c.wait()

    wv = [(w_v[pl.ds(32 * k, 16)], w_v[pl.ds(32 * k + 16, 16)])
          for k in range(8)]
    w14 = (wv[1][0] + wv[4][0], wv[1][1] + wv[4][1])
    iota = lax.iota(jnp.int32, 16)
    bvec = b_v[...]
    qb[...] = (row0[0, pl.ds(0, 16)] * wv[1][0]
               + row0[0, pl.ds(16, 16)] * wv[1][1])

    def rr(j, acc):
        return acc + plsc.load_gather(qb, [iota * 0 + j])

    rhov = lax.fori_loop(0, 16, rr, jnp.zeros((16,), jnp.float32))

    feats = ((f_u, 0), (f_v, 3), (f_p, 5), (f_c, 6), (f_cl, 6), (f_sc, 6),
             (f_in, 6), (f_r, 7), (f_mod, 2), (f_mf, 2), (f_sv, 2),
             (f_av, 2))

    def process(g, wbuf):
        def cbody(s, cnt):
            v = widx[pl.ds(g * ROWS_G + s * GS, GS)]
            return cnt + jnp.where(v == 0, 1, 0)

        cnt = lax.fori_loop(0, L, cbody, jnp.zeros((16,), jnp.int32))

        for o in (0, 8):
            z = []
            for j in range(8):
                z.append(wbuf[o + j, pl.ds(0, 16)] * w14[0])
                z.append(wbuf[o + j, pl.ds(16, 16)] * w14[1])

            def sbody(s, zt):
                new = []
                for j in range(8):
                    r = s * GS + o + j
                    new.append(zt[2 * j] + wbuf[r, pl.ds(0, 16)] * wv[1][0])
                    new.append(zt[2 * j + 1]
                               + wbuf[r, pl.ds(16, 16)] * wv[1][1])
                return tuple(new)

            zt = lax.fori_loop(1, L, sbody, tuple(z))

            for j in range(8):
                zlo, zhi = zt[2 * j], zt[2 * j + 1]
                sg = g * GS + o + j
                for buf, k in feats:
                    zlo = zlo + buf[sg, pl.ds(0, 16)] * wv[k][0]
                    zhi = zhi + buf[sg, pl.ds(16, 16)] * wv[k][1]
                zb[pl.ds((o + j) * EMB, 16)] = zlo
                zb[pl.ds((o + j) * EMB + 16, 16)] = zhi

        def rbody(j, s):
            return s + plsc.load_gather(zb, [iota * EMB + j])

        srow = lax.fori_loop(0, EMB, rbody, jnp.zeros((16,), jnp.float32))
        ov[pl.ds(g * GS, GS)] = srow - cnt.astype(jnp.float32) * rhov + bvec

    def gpair(gg, _):
        g0 = 2 * gg
        pltpu.make_async_copy(vemb_t.at[pl.ds(0, ROWS_G)], wbufa,
                              sem_a).wait()
        process(g0, wbufa)

        @pl.when(g0 + 2 < NG)
        def _():
            pltpu.async_copy(
                vemb_t.at[widx.at[pl.ds((g0 + 2) * ROWS_G, ROWS_G)]],
                wbufa, sem_a)

        pltpu.make_async_copy(vemb_t.at[pl.ds(0, ROWS_G)], wbufb,
                              sem_b).wait()
        process(g0 + 1, wbufb)

        @pl.when(g0 + 3 < NG)
        def _():
            pltpu.async_copy(
                vemb_t.at[widx.at[pl.ds((g0 + 3) * ROWS_G, ROWS_G)]],
                wbufb, sem_b)

        return 0

    lax.fori_loop(0, NG // 2, gpair, 0)
    pltpu.sync_copy(ov, out.at[pl.ds(base, SPW)])


def kernel(did, watch_vids, vid, prev, region, mod, mf, sver, aver, cid,
           class_id, second_class, is_intact,
           uemb_table, vemb_table, remb_table, mod_table, mf_table,
           sver_table, aver_table, cemb_table, class_table,
           second_class_table, intact_table, W, b):
    i32 = lambda x: x.astype(jnp.int32)
    watch_perm = (i32(watch_vids).reshape(NW, NG, GS, L)
                  .transpose(0, 1, 3, 2).reshape(-1))
    stk = jnp.stack([i32(x) for x in
                     (did, vid, prev, cid, region, mod, mf, sver, aver,
                      class_id, second_class, is_intact)])
    mesh = plsc.VectorSubcoreMesh(core_axis_name="c", subcore_axis_name="s")
    scratch = [
        pltpu.VMEM((SPW * L,), jnp.int32),
        pltpu.VMEM((NF, SPW), jnp.int32),
        pltpu.VMEM((ROWS_G, EMB), jnp.float32),
        pltpu.VMEM((ROWS_G, EMB), jnp.float32),
    ] + [pltpu.VMEM((SPW, EMB), jnp.float32)] * NF + [
        pltpu.VMEM((8 * EMB,), jnp.float32),
        pltpu.VMEM((16,), jnp.float32),
        pltpu.VMEM((1, EMB), jnp.float32),
        pltpu.VMEM((GS * EMB,), jnp.float32),
        pltpu.VMEM((16,), jnp.float32),
        pltpu.VMEM((SPW,), jnp.float32),
        pltpu.SemaphoreType.DMA,
        pltpu.SemaphoreType.DMA,
        pltpu.SemaphoreType.DMA,
        pltpu.SemaphoreType.DMA,
    ]
    run = pl.kernel(_body,
                    out_type=jax.ShapeDtypeStruct((B,), jnp.float32),
                    mesh=mesh, scratch_types=scratch,
                    compiler_params=pltpu.CompilerParams(
                        needs_layout_passes=False,
                        use_tc_tiling_on_sc=False))
    return run(watch_perm, stk,
               uemb_table, vemb_table, remb_table, mod_table, mf_table,
               sver_table, aver_table, cemb_table, class_table,
               second_class_table, intact_table,
               W.reshape(-1).astype(jnp.float32),
               jnp.broadcast_to(b.astype(jnp.float32).reshape(()), (16,)))
